# baseline (speedup 1.0000x reference)
import functools

import jax
import jax.numpy as jnp
from jax import lax
from jax.experimental import pallas as pl
from jax.experimental.pallas import tpu as pltpu
from jax.experimental.pallas import tpu_sc as plsc

NC, NS, L = 2, 16, 16
B, T, P, A = 16, 800, 543, 3
NP = 109
F = 2 * NP
FP = 224
NG = FP // L
ROW = P * A
RPT = (B * T) // (NC * NS)
CH = 8
NCHUNK = RPT // CH
SEG = (0, 80, 134, 176, 218)


def _seg_of_group(g):
    out = []
    for s in range(4):
        lo, hi = SEG[s], SEG[s + 1]
        if lo < (g + 1) * L and hi > g * L:
            out.append((s, lo <= g * L and hi >= (g + 1) * L, lo, hi))
    return out


def _fg_body(x_hbm, idx_hbm, out_hbm, kps, stage, idxv, acc, pacc, consts, shared):
    s = lax.axis_index("s")
    c = lax.axis_index("c")
    batch = c * (B // NC) + s // 2
    half = s % 2
    base_row = batch * T + half * RPT
    iota = lax.iota(jnp.int32, L)
    zero = jnp.zeros((L,), jnp.float32)

    pltpu.sync_copy(idx_hbm, idxv)
    for k in range(12):
        acc[k] = zero

    def chunk_body(ci, _):
        r0 = base_row + ci * CH
        pltpu.sync_copy(x_hbm.at[pl.ds(r0 * ROW, CH * ROW)], stage)

        def row_body(r, _):
            rloc = ci * CH + r
            rbase = jnp.full((L,), r * ROW, jnp.int32)
            for g in range(NG):
                iv = idxv[pl.ds(g * L, L)]
                v = plsc.load_gather(stage, [rbase + iv])
                kps[pl.ds(rloc * F + g * L, L)] = v
                nan = v != v
                v0 = jnp.where(nan, 0.0, v)
                f1 = jnp.where(nan, 0.0, 1.0)
                for sg, full, lo, hi in _seg_of_group(g):
                    if full:
                        vz, fz = v0, f1
                    else:
                        j = g * L + iota
                        m = (j >= lo) & (j < hi)
                        vz = jnp.where(m, v0, 0.0)
                        fz = jnp.where(m, f1, 0.0)
                    acc[sg] = acc[sg] + vz
                    acc[4 + sg] = acc[4 + sg] + vz * vz
                    acc[8 + sg] = acc[8 + sg] + fz
            return 0

        lax.fori_loop(0, CH, row_body, 0)
        return 0

    lax.fori_loop(0, NCHUNK, chunk_body, 0)

    pltpu.sync_copy(acc, shared.at[s])
    plsc.subcore_barrier()
    pltpu.sync_copy(shared.at[s ^ 1], pacc)

    meanv, invv = [], []
    for sg in range(4):
        smv = jnp.full((L,), jnp.sum(acc[sg] + pacc[sg]), jnp.float32)
        sm2v = jnp.full((L,), jnp.sum(acc[4 + sg] + pacc[4 + sg]), jnp.float32)
        cnv = jnp.full((L,), jnp.sum(acc[8 + sg] + pacc[8 + sg]), jnp.float32)
        cnv = jnp.maximum(cnv, 1.0)
        mean = smv / cnv
        vv = jnp.maximum(sm2v / cnv - mean * mean, 0.0) + 1e-6
        bits = plsc.bitcast(vv, jnp.int32)
        y = plsc.bitcast(jnp.int32(0x5F3759DF) - (bits >> 1), jnp.float32)
        for _ in range(4):
            y = y * (1.5 - 0.5 * vv * y * y)
        meanv.append(mean)
        invv.append(y)

    for g in range(NG):
        segs = _seg_of_group(g)
        m, y = meanv[segs[0][0]], invv[segs[0][0]]
        for sg, _, lo, _ in segs[1:]:
            sel = (g * L + iota) >= lo
            m = jnp.where(sel, meanv[sg], m)
            y = jnp.where(sel, invv[sg], y)
        consts[g] = m
        consts[NG + g] = y

    def norm_body(rloc, _):
        for g in range(NG):
            off = rloc * F + g * L
            v = kps[pl.ds(off, L)]
            o = (v - consts[g]) * consts[NG + g]
            o = jnp.where(v != v, 0.0, o)
            if g == NG - 1:
                o = jnp.where(g * L + iota < F, o, v)
            kps[pl.ds(off, L)] = o
        return 0

    lax.fori_loop(0, RPT, norm_body, 0)

    pltpu.sync_copy(kps.at[pl.ds(0, RPT * F)], out_hbm.at[pl.ds(base_row * F, RPT * F)])


_fg_call = functools.partial(
    pl.kernel,
    out_type=jax.ShapeDtypeStruct((B * T * F,), jnp.float32),
    mesh=plsc.VectorSubcoreMesh(
        core_axis_name="c", subcore_axis_name="s", num_cores=NC, num_subcores=NS
    ),
    compiler_params=pltpu.CompilerParams(
        use_tc_tiling_on_sc=False, needs_layout_passes=False
    ),
    scratch_types=[
        pltpu.VMEM((RPT * F + L,), jnp.float32),
        pltpu.VMEM((CH * ROW,), jnp.float32),
        pltpu.VMEM((FP,), jnp.int32),
        pltpu.VMEM((12, L), jnp.float32),
        pltpu.VMEM((12, L), jnp.float32),
        pltpu.VMEM((2 * NG, L), jnp.float32),
        pltpu.VMEM_SHARED((NS, 12, L), jnp.float32),
    ],
)(_fg_body)


def kernel(x, point_arr):
    pa = point_arr.astype(jnp.int32)
    idx = 3 * jnp.repeat(pa, 2) + jnp.tile(jnp.arange(2, dtype=jnp.int32), NP)
    idx_pad = jnp.concatenate([idx, jnp.zeros((FP - F,), jnp.int32)])
    out = _fg_call(x.reshape(B * T * ROW), idx_pad)
    return out.reshape(B, T, NP, 2)


# speedup vs baseline: 247.4203x; 247.4203x over previous
import functools

import jax
import jax.numpy as jnp
from jax import lax
from jax.experimental import pallas as pl
from jax.experimental.pallas import tpu as pltpu
from jax.experimental.pallas import tpu_sc as plsc

NC, NS, L = 2, 16, 16
B, T, P, A = 16, 800, 543, 3
NP = 109
NBH, NTH = B // 8, 7
PLW = NBH * NTH * 8 * 128
MAXK = 4


def _fire_plane(xt, planes, sem, par, p):
    for a in range(2):
        yield xt.at[p, a], planes.at[par, a], sem


def _body(xt, pa_hbm, out, planes, ostage, idxv, accv, accf, rec, statv,
          insem, outsem):
    s = lax.axis_index("s")
    c = lax.axis_index("c")
    iota = lax.iota(jnp.int32, L)
    zero = jnp.zeros((L,), jnp.float32)

    start0 = 4 * s - jnp.maximum(0, s - 13)
    cnt0 = 4 - (s >= 13).astype(jnp.int32)
    start = jnp.where(c == 0, start0, 3 * s)
    cnt = jnp.where(c == 0, cnt0, 3)

    def out_idx(kk):
        j = start + kk
        i0 = j + jnp.where(j >= 40, 48, 0)
        return jnp.where(c == 0, i0, 40 + j)

    def src_point(i):
        base = (i // 16) * 16
        chunk = idxv[pl.ds(base, 16)]
        return jnp.sum(jnp.where(iota == i % 16, chunk, 0))

    def seg_of(i):
        return ((i >= 40).astype(jnp.int32) + (i >= 67).astype(jnp.int32)
                + (i >= 88).astype(jnp.int32))

    pltpu.sync_copy(pa_hbm, idxv.at[pl.ds(0, NP)])
    for k in range(16):
        accv[k] = zero


    def fire(p, par):
        for src, dst, sm in _fire_plane(xt, planes, insem, par, p):
            pltpu.async_copy(src, dst, sm)

    def drain(p, par):
        for src, dst, sm in _fire_plane(xt, planes, insem, par, p):
            pltpu.make_async_copy(src, dst, sm).wait()

    chunks = [(th, cc) for th in range(NTH) for cc in range(8 if th < 6 else 2)]

    def stats_body(kk, _):
        @pl.when(kk < cnt)
        def _():
            par = kk % 2
            i = out_idx(kk)
            sg = seg_of(i)
            fire(src_point(i), par)
            drain(src_point(i), par)

            def b_body(b, _):
                rs = rq = rc = zero
                for a in range(2):
                    for th, cc in chunks:
                        v = planes[par, a, b, pl.ds(th * 128 + cc * 16, 16)]
                        nan = v != v
                        v0 = jnp.where(nan, 0.0, v)
                        rs = rs + v0
                        rq = rq + v0 * v0
                        rc = rc + jnp.where(nan, 0.0, 1.0)
                onb = iota == b
                accv[3 * sg] = accv[3 * sg] + jnp.where(onb, jnp.sum(rs), 0.0)
                accv[3 * sg + 1] = accv[3 * sg + 1] + jnp.where(onb, jnp.sum(rq), 0.0)
                accv[3 * sg + 2] = accv[3 * sg + 2] + jnp.where(onb, jnp.sum(rc), 0.0)
                return 0

            lax.fori_loop(0, B, b_body, 0)
        return 0

    lax.fori_loop(0, MAXK, stats_body, 0)

    for k in range(16):
        accf[pl.ds(k * 16, 16)] = accv[k]
    i_rec = 40 * c
    pltpu.sync_copy(accf, out.at[s, i_rec, 0, pl.ds(0, 256)])
    plsc.subcore_barrier()

    sms = [zero] * 4
    sqs = [zero] * 4
    cns = [zero] * 4
    for t in range(NS):
        pltpu.sync_copy(out.at[t, i_rec, 0, pl.ds(0, 256)], rec)
        for sg in range(4):
            sms[sg] = sms[sg] + rec[pl.ds(3 * sg * 16, 16)]
            sqs[sg] = sqs[sg] + rec[pl.ds((3 * sg + 1) * 16, 16)]
            cns[sg] = cns[sg] + rec[pl.ds((3 * sg + 2) * 16, 16)]
    plsc.subcore_barrier()

    for sg in range(4):
        sm = sms[sg]
        sq = sqs[sg]
        cn = jnp.maximum(cns[sg], 1.0)
        mean = sm / cn
        vv = jnp.maximum(sq / cn - mean * mean, 0.0) + 1e-6
        bits = plsc.bitcast(vv, jnp.int32)
        y = plsc.bitcast(jnp.int32(0x5F3759DF) - (bits >> 1), jnp.float32)
        for _ in range(4):
            y = y * (1.5 - 0.5 * vv * y * y)
        statv[sg] = mean
        statv[4 + sg] = y

    def norm_body(kk, _):
        @pl.when(kk < cnt)
        def _():
            par = kk % 2
            i = out_idx(kk)
            sg = seg_of(i)
            mvec = statv[sg]
            ivec = statv[4 + sg]
            fire(src_point(i), par)
            drain(src_point(i), par)

            def b_body(b, _):
                mb = jnp.sum(jnp.where(iota == b, mvec, 0.0))
                ib = jnp.sum(jnp.where(iota == b, ivec, 0.0))
                for a in range(2):
                    for th, cc in chunks:
                        t0 = th * 128 + cc * 16
                        v = planes[par, a, b, pl.ds(t0, 16)]
                        o = (v - mb) * ib
                        o = jnp.where(v != v, 0.0, o)
                        ostage[0, b, a, pl.ds(t0, 16)] = o
                return 0

            lax.fori_loop(0, B, b_body, 0)
            pltpu.sync_copy(ostage.at[0], out.at[:, i])
        return 0

    lax.fori_loop(0, MAXK, norm_body, 0)


_fg_call = functools.partial(
    pl.kernel,
    out_type=jax.ShapeDtypeStruct((B, NP, 2, T), jnp.float32),
    mesh=plsc.VectorSubcoreMesh(
        core_axis_name="c", subcore_axis_name="s", num_cores=NC, num_subcores=NS
    ),
    compiler_params=pltpu.CompilerParams(needs_layout_passes=False),
    scratch_types=[
        pltpu.VMEM((2, 2, B, T), jnp.float32),
        pltpu.VMEM((1, B, 2, T), jnp.float32),
        pltpu.VMEM((112,), jnp.int32),
        pltpu.VMEM((16, L), jnp.float32),
        pltpu.VMEM((256,), jnp.float32),
        pltpu.VMEM((256,), jnp.float32),
        pltpu.VMEM((8, L), jnp.float32),
        pltpu.SemaphoreType.DMA,
        pltpu.SemaphoreType.DMA,
    ],
)(_body)


def kernel(x, point_arr):
    xt = jnp.transpose(x, (2, 3, 0, 1))
    o = _fg_call(xt, point_arr.astype(jnp.int32))
    return jnp.transpose(o, (0, 3, 1, 2))


# speedup vs baseline: 273.6737x; 1.1061x over previous
import functools

import jax
import jax.numpy as jnp
from jax import lax
from jax.experimental import pallas as pl
from jax.experimental.pallas import tpu as pltpu
from jax.experimental.pallas import tpu_sc as plsc

NC, NS, L = 2, 16, 16
B, T, P, A = 16, 800, 543, 3
NP = 109
NBH, NTH = B // 8, 7
PLW = NBH * NTH * 8 * 128
MAXK = 4


def _fire_plane(xt, planes, sem, par, p):
    for a in range(2):
        yield xt.at[p, a], planes.at[par, a], sem


def _body(xt, pa_hbm, out, planes, ostage, idxv, accv, accf, rec, statv,
          insem, outsem):
    s = lax.axis_index("s")
    c = lax.axis_index("c")
    iota = lax.iota(jnp.int32, L)
    zero = jnp.zeros((L,), jnp.float32)

    start0 = 4 * s - jnp.maximum(0, s - 13)
    cnt0 = 4 - (s >= 13).astype(jnp.int32)
    start = jnp.where(c == 0, start0, 3 * s)
    cnt = jnp.where(c == 0, cnt0, 3)

    def out_idx(kk):
        j = start + kk
        i0 = j + jnp.where(j >= 40, 48, 0)
        return jnp.where(c == 0, i0, 40 + j)

    def src_point(i):
        base = (i // 16) * 16
        chunk = idxv[pl.ds(base, 16)]
        return jnp.sum(jnp.where(iota == i % 16, chunk, 0))

    def seg_of(i):
        return ((i >= 40).astype(jnp.int32) + (i >= 67).astype(jnp.int32)
                + (i >= 88).astype(jnp.int32))

    pltpu.sync_copy(pa_hbm, idxv.at[pl.ds(0, NP)])
    for k in range(16):
        accv[k] = zero


    def fire(p, par):
        for src, dst, sm in _fire_plane(xt, planes, insem, par, p):
            pltpu.async_copy(src, dst, sm)

    def drain(p, par):
        for src, dst, sm in _fire_plane(xt, planes, insem, par, p):
            pltpu.make_async_copy(src, dst, sm).wait()

    chunks = [(th, cc) for th in range(NTH) for cc in range(8 if th < 6 else 2)]

    @pl.when(cnt > 0)
    def _():
        fire(src_point(out_idx(0)), 0)

    def stats_body(kk, _):
        @pl.when(kk < cnt)
        def _():
            par = kk % 2
            i = out_idx(kk)
            sg = seg_of(i)

            @pl.when(kk + 1 < cnt)
            def _():
                fire(src_point(out_idx(kk + 1)), (kk + 1) % 2)

            drain(src_point(i), par)

            def b_body(b, _):
                rs = rq = rc = zero
                for a in range(2):
                    for th, cc in chunks:
                        v = planes[par, a, b, pl.ds(th * 128 + cc * 16, 16)]
                        nan = v != v
                        v0 = jnp.where(nan, 0.0, v)
                        rs = rs + v0
                        rq = rq + v0 * v0
                        rc = rc + jnp.where(nan, 0.0, 1.0)
                onb = iota == b
                accv[3 * sg] = accv[3 * sg] + jnp.where(onb, jnp.sum(rs), 0.0)
                accv[3 * sg + 1] = accv[3 * sg + 1] + jnp.where(onb, jnp.sum(rq), 0.0)
                accv[3 * sg + 2] = accv[3 * sg + 2] + jnp.where(onb, jnp.sum(rc), 0.0)
                return 0

            lax.fori_loop(0, B, b_body, 0)
        return 0

    lax.fori_loop(0, MAXK, stats_body, 0)

    for k in range(16):
        accf[pl.ds(k * 16, 16)] = accv[k]
    i_rec = 40 * c
    pltpu.sync_copy(accf, out.at[s, i_rec, 0, pl.ds(0, 256)])
    plsc.subcore_barrier()

    sms = [zero] * 4
    sqs = [zero] * 4
    cns = [zero] * 4
    for t in range(NS):
        pltpu.sync_copy(out.at[t, i_rec, 0, pl.ds(0, 256)], rec)
        for sg in range(4):
            sms[sg] = sms[sg] + rec[pl.ds(3 * sg * 16, 16)]
            sqs[sg] = sqs[sg] + rec[pl.ds((3 * sg + 1) * 16, 16)]
            cns[sg] = cns[sg] + rec[pl.ds((3 * sg + 2) * 16, 16)]
    plsc.subcore_barrier()

    for sg in range(4):
        sm = sms[sg]
        sq = sqs[sg]
        cn = jnp.maximum(cns[sg], 1.0)
        mean = sm / cn
        vv = jnp.maximum(sq / cn - mean * mean, 0.0) + 1e-6
        bits = plsc.bitcast(vv, jnp.int32)
        y = plsc.bitcast(jnp.int32(0x5F3759DF) - (bits >> 1), jnp.float32)
        for _ in range(4):
            y = y * (1.5 - 0.5 * vv * y * y)
        statv[sg] = mean
        statv[4 + sg] = y

    @pl.when(cnt > 0)
    def _():
        fire(src_point(out_idx(0)), 0)

    def norm_body(kk, _):
        @pl.when(kk < cnt)
        def _():
            par = kk % 2
            i = out_idx(kk)
            sg = seg_of(i)
            mvec = statv[sg]
            ivec = statv[4 + sg]

            @pl.when(kk + 1 < cnt)
            def _():
                fire(src_point(out_idx(kk + 1)), (kk + 1) % 2)

            drain(src_point(i), par)

            def b_body(b, _):
                mb = jnp.sum(jnp.where(iota == b, mvec, 0.0))
                ib = jnp.sum(jnp.where(iota == b, ivec, 0.0))
                for a in range(2):
                    for th, cc in chunks:
                        t0 = th * 128 + cc * 16
                        v = planes[par, a, b, pl.ds(t0, 16)]
                        o = (v - mb) * ib
                        o = jnp.where(v != v, 0.0, o)
                        ostage[0, b, a, pl.ds(t0, 16)] = o
                return 0

            lax.fori_loop(0, B, b_body, 0)
            pltpu.sync_copy(ostage.at[0], out.at[:, i])
        return 0

    lax.fori_loop(0, MAXK, norm_body, 0)


_fg_call = functools.partial(
    pl.kernel,
    out_type=jax.ShapeDtypeStruct((B, NP, 2, T), jnp.float32),
    mesh=plsc.VectorSubcoreMesh(
        core_axis_name="c", subcore_axis_name="s", num_cores=NC, num_subcores=NS
    ),
    compiler_params=pltpu.CompilerParams(needs_layout_passes=False),
    scratch_types=[
        pltpu.VMEM((2, 2, B, T), jnp.float32),
        pltpu.VMEM((1, B, 2, T), jnp.float32),
        pltpu.VMEM((112,), jnp.int32),
        pltpu.VMEM((16, L), jnp.float32),
        pltpu.VMEM((256,), jnp.float32),
        pltpu.VMEM((256,), jnp.float32),
        pltpu.VMEM((8, L), jnp.float32),
        pltpu.SemaphoreType.DMA,
        pltpu.SemaphoreType.DMA,
    ],
)(_body)


def kernel(x, point_arr):
    xt = jnp.transpose(x, (2, 3, 0, 1))
    o = _fg_call(xt, point_arr.astype(jnp.int32))
    return jnp.transpose(o, (0, 3, 1, 2))
